# baseline (speedup 1.0000x reference)
import functools

import jax
import jax.numpy as jnp
from jax import lax
from jax.experimental import pallas as pl
from jax.experimental.pallas import tpu as pltpu
from jax.experimental.pallas import tpu_sc as plsc

N = 10000
E = 160000
NC, NS = 2, 16
NW = NC * NS
C = 128
CHT = 40
EPW = E // NW
NFULL = EPW // C
TAIL = EPW - NFULL * C

N_PAD = 10240
RPT = N_PAD // NS
ZR = 160

_mesh = lambda: plsc.VectorSubcoreMesh(core_axis_name="c", subcore_axis_name="s")
_SC_LINEAR = pltpu.CompilerParams(use_tc_tiling_on_sc=False)


def _stage_idx(src_hbm, dst_hbm, sidx, didx, base, sem, c=C, cht=CHT):
    nfull = EPW // c
    iota = lax.iota(jnp.int32, 16)
    for j in range(nfull, cht):
        for q in range(c // 16):
            sidx[j, pl.ds(q * 16, 16)] = iota
            didx[j, pl.ds(q * 16, 16)] = iota + N
    descs = []
    for j in range(nfull):
        descs.append(pltpu.async_copy(
            src_hbm.at[pl.ds(base + j * c, c)], sidx.at[j], sem))
        descs.append(pltpu.async_copy(
            dst_hbm.at[pl.ds(base + j * c, c)], didx.at[j], sem))
    descs.append(pltpu.async_copy(
        src_hbm.at[pl.ds(base + nfull * c, TAIL)],
        sidx.at[nfull, pl.ds(0, TAIL)], sem))
    descs.append(pltpu.async_copy(
        dst_hbm.at[pl.ds(base + nfull * c, TAIL)],
        didx.at[nfull, pl.ds(0, TAIL)], sem))
    for d in descs:
        d.wait()


@functools.partial(
    pl.kernel,
    out_type=jax.ShapeDtypeStruct((NC * N_PAD,), jnp.float32),
    mesh=_mesh(),
    scratch_types=[
        pltpu.VMEM_SHARED((N_PAD,), jnp.float32),
        pltpu.VMEM((CHT, C), jnp.int32),
        pltpu.VMEM((CHT, C), jnp.int32),
        pltpu.VMEM((C,), jnp.float32),
        pltpu.VMEM((RPT,), jnp.float32),
    ] + [pltpu.SemaphoreType.DMA] * 8,
)
def _deg(src_hbm, dst_hbm, out_hbm, acc, didx, sidx, ones_v, zbuf, *sems):
    cid = lax.axis_index("c")
    sid = lax.axis_index("s")
    for i in range(C // 16):
        ones_v[pl.ds(i * 16, 16)] = jnp.full((16,), 1.0, jnp.float32)
    for i in range(RPT // 16):
        zbuf[pl.ds(i * 16, 16)] = jnp.zeros((16,), jnp.float32)
    base = (cid * NS + sid) * EPW
    _stage_idx(src_hbm, dst_hbm, sidx, didx, base, sems[0])
    pltpu.sync_copy(zbuf, acc.at[pl.ds(sid * RPT, RPT)])
    plsc.subcore_barrier()

    @pl.loop(0, CHT // 8)
    def _(r):
        c0 = r * 8
        descs = [
            pltpu.async_copy(ones_v, acc.at[didx.at[c0 + b]], sems[b], add=True)
            for b in range(8)
        ]
        for d in descs:
            d.wait()

    plsc.subcore_barrier()
    pltpu.sync_copy(acc.at[pl.ds(sid * RPT, RPT)],
                    out_hbm.at[pl.ds(cid * N_PAD + sid * RPT, RPT)])


def _prop_loop(g_hbm, acc, sidx, didx, rows, gsems, ssems, K, cht=CHT):
    ROUNDS = cht // K
    for b in range(K):
        pltpu.async_copy(g_hbm.at[sidx.at[b]], rows.at[b], gsems[b])

    @pl.loop(0, ROUNDS - 1)
    def _(r):
        c0 = r * K
        sds = []
        for b in range(K):
            pltpu.make_async_copy(g_hbm.at[sidx.at[c0 + b]], rows.at[b],
                                  gsems[b]).wait()
            sds.append(pltpu.async_copy(rows.at[b], acc.at[didx.at[c0 + b]],
                                        ssems[b], add=True))
        for b in range(K):
            sds[b].wait()
            pltpu.async_copy(g_hbm.at[sidx.at[c0 + K + b]], rows.at[b],
                             gsems[b])

    c0 = (ROUNDS - 1) * K
    sds = []
    for b in range(K):
        pltpu.make_async_copy(g_hbm.at[sidx.at[c0 + b]], rows.at[b],
                              gsems[b]).wait()
        sds.append(pltpu.async_copy(rows.at[b], acc.at[didx.at[c0 + b]],
                                    ssems[b], add=True))
    for d in sds:
        d.wait()


_K64 = 8
_C64 = 64
_CHT64 = 80
_K128 = 2
_C128 = 128
_CHT128 = 40


@functools.partial(
    pl.kernel,
    out_type=jax.ShapeDtypeStruct((NC * N_PAD, 64), jnp.float32),
    mesh=_mesh(),
    scratch_types=[
        pltpu.VMEM_SHARED((N_PAD, 64), jnp.float32),
        pltpu.VMEM((_CHT64, _C64), jnp.int32),
        pltpu.VMEM((_CHT64, _C64), jnp.int32),
        pltpu.VMEM((_K64, _C64, 64), jnp.float32),
        pltpu.VMEM((ZR, 64), jnp.float32),
    ] + [pltpu.SemaphoreType.DMA] * (2 * _K64),
    compiler_params=_SC_LINEAR,
)
def _prop64(g_hbm, src_hbm, dst_hbm, out_hbm, acc, sidx, didx, rows, zbuf,
            *sems):
    K = _K64
    cid = lax.axis_index("c")
    sid = lax.axis_index("s")

    @pl.loop(0, ZR)
    def _(r):
        for q in range(4):
            zbuf[r, pl.ds(q * 16, 16)] = jnp.zeros((16,), jnp.float32)

    base = (cid * NS + sid) * EPW
    _stage_idx(src_hbm, dst_hbm, sidx, didx, base, sems[0], _C64, _CHT64)
    zd = [
        pltpu.async_copy(zbuf, acc.at[pl.ds(sid * RPT + z * ZR, ZR)], sems[z])
        for z in range(RPT // ZR)
    ]
    for d in zd:
        d.wait()
    plsc.subcore_barrier()

    _prop_loop(g_hbm, acc, sidx, didx, rows, sems[:K], sems[K:], K, _CHT64)

    plsc.subcore_barrier()
    pltpu.sync_copy(acc.at[pl.ds(sid * RPT, RPT)],
                    out_hbm.at[pl.ds(cid * N_PAD + sid * RPT, RPT)])


@functools.partial(
    pl.kernel,
    out_type=jax.ShapeDtypeStruct((NC * N_PAD, 128), jnp.float32),
    mesh=_mesh(),
    scratch_types=[
        pltpu.VMEM_SHARED((N_PAD, 128), jnp.float32),
        pltpu.VMEM((_CHT128, _C128), jnp.int32),
        pltpu.VMEM((_CHT128, _C128), jnp.int32),
        pltpu.VMEM((_K128, _C128, 128), jnp.float32),
    ] + [pltpu.SemaphoreType.DMA] * (2 * _K128 + 1),
)
def _prop128(g_hbm, src_hbm, dst_hbm, out_hbm, acc, sidx, didx, rows, *sems):
    K = _K128
    gsems, ssems, zsem = sems[:K], sems[K:2 * K], sems[2 * K]
    cid = lax.axis_index("c")
    sid = lax.axis_index("s")

    @pl.loop(0, _C128)
    def _(r):
        for q in range(8):
            rows[0, r, pl.ds(q * 16, 16)] = jnp.zeros((16,), jnp.float32)

    base = (cid * NS + sid) * EPW
    _stage_idx(src_hbm, dst_hbm, sidx, didx, base, zsem, _C128, _CHT128)
    zd = [
        pltpu.async_copy(rows.at[0],
                         acc.at[pl.ds(sid * RPT + z * _C128, _C128)], zsem)
        for z in range(RPT // _C128)
    ]
    for d in zd:
        d.wait()
    plsc.subcore_barrier()

    _prop_loop(g_hbm, acc, sidx, didx, rows, gsems, ssems, K, _CHT128)

    plsc.subcore_barrier()
    pltpu.sync_copy(acc.at[pl.ds(sid * RPT, RPT)],
                    out_hbm.at[pl.ds(cid * N_PAD + sid * RPT, RPT)])


R = 1024
GRID = N_PAD // R


def _tc_first(x, W, c0, c1):
    Din, Dout = W.shape

    def body(x_ref, w_ref, c0_ref, c1_ref, g_ref, dinv_ref):
        h = jnp.dot(x_ref[...], w_ref[...], preferred_element_type=jnp.float32)
        dinv = lax.rsqrt(c0_ref[...] + c1_ref[...] + 1.0)
        dinv_ref[...] = dinv
        g_ref[...] = h * dinv

    return pl.pallas_call(
        body,
        grid=(GRID,),
        in_specs=[
            pl.BlockSpec((R, Din), lambda i: (i, 0)),
            pl.BlockSpec((Din, Dout), lambda i: (0, 0)),
            pl.BlockSpec((R, 1), lambda i: (i, 0)),
            pl.BlockSpec((R, 1), lambda i: (i, 0)),
        ],
        out_specs=[
            pl.BlockSpec((R, Dout), lambda i: (i, 0)),
            pl.BlockSpec((R, 1), lambda i: (i, 0)),
        ],
        out_shape=[
            jax.ShapeDtypeStruct((N, Dout), jnp.float32),
            jax.ShapeDtypeStruct((N_PAD, 1), jnp.float32),
        ],
    )(x, W, c0, c1)


def _tc_mid(acc, g, dinv, b, W):
    Din, Dout = W.shape

    def body(a0_ref, a1_ref, g_ref, dv_ref, b_ref, w_ref, o_ref):
        dcol = dv_ref[...]
        s = dcol * (a0_ref[...] + a1_ref[...] + g_ref[...]) + b_ref[...]
        act = jnp.where(s >= 0, s, 0.2 * s)
        h = jnp.dot(act, w_ref[...], preferred_element_type=jnp.float32)
        o_ref[...] = h * dcol

    return pl.pallas_call(
        body,
        grid=(GRID,),
        in_specs=[
            pl.BlockSpec((R, Din), lambda i: (i, 0)),
            pl.BlockSpec((R, Din), lambda i: (i + GRID, 0)),
            pl.BlockSpec((R, Din), lambda i: (i, 0)),
            pl.BlockSpec((R, 1), lambda i: (i, 0)),
            pl.BlockSpec((1, Din), lambda i: (0, 0)),
            pl.BlockSpec((Din, Dout), lambda i: (0, 0)),
        ],
        out_specs=pl.BlockSpec((R, Dout), lambda i: (i, 0)),
        out_shape=jax.ShapeDtypeStruct((N, Dout), jnp.float32),
    )(acc, acc, g, dinv, b, W)


def _tc_last(acc, g, dinv, b):
    F = g.shape[1]

    def body(a0_ref, a1_ref, g_ref, dv_ref, b_ref, o_ref):
        o_ref[...] = (dv_ref[...] * (a0_ref[...] + a1_ref[...] + g_ref[...])
                      + b_ref[...])

    return pl.pallas_call(
        body,
        grid=(GRID,),
        in_specs=[
            pl.BlockSpec((R, F), lambda i: (i, 0)),
            pl.BlockSpec((R, F), lambda i: (i + GRID, 0)),
            pl.BlockSpec((R, F), lambda i: (i, 0)),
            pl.BlockSpec((R, 1), lambda i: (i, 0)),
            pl.BlockSpec((1, F), lambda i: (0, 0)),
        ],
        out_specs=pl.BlockSpec((R, F), lambda i: (i, 0)),
        out_shape=jax.ShapeDtypeStruct((N, F), jnp.float32),
    )(acc, acc, g, dinv, b)


def kernel(x, edge_index, W1, b1, W2, b2, W3, b3):
    ei = edge_index.astype(jnp.int32)
    src, dst = ei[0], ei[1]

    cnt = _deg(src, dst)
    c0 = cnt[:N_PAD].reshape(N_PAD, 1)
    c1 = cnt[N_PAD:].reshape(N_PAD, 1)

    g1, dinv = _tc_first(x, W1, c0, c1)
    acc = _prop128(g1, src, dst)
    g2 = _tc_mid(acc, g1, dinv, b1.reshape(1, -1), W2)
    acc = _prop64(g2, src, dst)
    g3 = _tc_mid(acc, g2, dinv, b2.reshape(1, -1), W3)
    acc = _prop64(g3, src, dst)
    return _tc_last(acc, g3, dinv, b3.reshape(1, -1))


# speedup vs baseline: 1.0162x; 1.0162x over previous
import functools

import jax
import jax.numpy as jnp
from jax import lax
from jax.experimental import pallas as pl
from jax.experimental.pallas import tpu as pltpu
from jax.experimental.pallas import tpu_sc as plsc

N = 10000
E = 160000
NC, NS = 2, 16
NW = NC * NS
C = 128
CHT = 40
EPW = E // NW
NFULL = EPW // C
TAIL = EPW - NFULL * C

N_PAD = 10240
RPT = N_PAD // NS
ZR = 160

_mesh = lambda: plsc.VectorSubcoreMesh(core_axis_name="c", subcore_axis_name="s")
_SC_LINEAR = pltpu.CompilerParams(use_tc_tiling_on_sc=False)


def _stage_idx(src_hbm, dst_hbm, sidx, didx, base, sem, c=C, cht=CHT):
    nfull = EPW // c
    iota = lax.iota(jnp.int32, 16)
    for j in range(nfull, cht):
        for q in range(c // 16):
            sidx[j, pl.ds(q * 16, 16)] = iota
            didx[j, pl.ds(q * 16, 16)] = iota + N
    descs = []
    for j in range(nfull):
        descs.append(pltpu.async_copy(
            src_hbm.at[pl.ds(base + j * c, c)], sidx.at[j], sem))
        descs.append(pltpu.async_copy(
            dst_hbm.at[pl.ds(base + j * c, c)], didx.at[j], sem))
    descs.append(pltpu.async_copy(
        src_hbm.at[pl.ds(base + nfull * c, TAIL)],
        sidx.at[nfull, pl.ds(0, TAIL)], sem))
    descs.append(pltpu.async_copy(
        dst_hbm.at[pl.ds(base + nfull * c, TAIL)],
        didx.at[nfull, pl.ds(0, TAIL)], sem))
    for d in descs:
        d.wait()


@functools.partial(
    pl.kernel,
    out_type=jax.ShapeDtypeStruct((NC * N_PAD,), jnp.float32),
    mesh=_mesh(),
    scratch_types=[
        pltpu.VMEM_SHARED((N_PAD,), jnp.float32),
        pltpu.VMEM((CHT, C), jnp.int32),
        pltpu.VMEM((CHT, C), jnp.int32),
        pltpu.VMEM((C,), jnp.float32),
        pltpu.VMEM((RPT,), jnp.float32),
    ] + [pltpu.SemaphoreType.DMA] * 8,
)
def _deg(src_hbm, dst_hbm, out_hbm, acc, didx, sidx, ones_v, zbuf, *sems):
    cid = lax.axis_index("c")
    sid = lax.axis_index("s")
    for i in range(C // 16):
        ones_v[pl.ds(i * 16, 16)] = jnp.full((16,), 1.0, jnp.float32)
    for i in range(RPT // 16):
        zbuf[pl.ds(i * 16, 16)] = jnp.zeros((16,), jnp.float32)
    base = (cid * NS + sid) * EPW
    _stage_idx(src_hbm, dst_hbm, sidx, didx, base, sems[0])
    pltpu.sync_copy(zbuf, acc.at[pl.ds(sid * RPT, RPT)])
    plsc.subcore_barrier()

    @pl.loop(0, CHT // 8)
    def _(r):
        c0 = r * 8
        descs = [
            pltpu.async_copy(ones_v, acc.at[didx.at[c0 + b]], sems[b], add=True)
            for b in range(8)
        ]
        for d in descs:
            d.wait()

    plsc.subcore_barrier()
    pltpu.sync_copy(acc.at[pl.ds(sid * RPT, RPT)],
                    out_hbm.at[pl.ds(cid * N_PAD + sid * RPT, RPT)])


def _prop_loop(g_hbm, acc, sidx, didx, rows, gsems, ssems, K, cht=CHT):
    ROUNDS = cht // K
    for b in range(K):
        pltpu.async_copy(g_hbm.at[sidx.at[b]], rows.at[b], gsems[b])

    @pl.loop(0, ROUNDS - 1)
    def _(r):
        c0 = r * K
        sds = []
        for b in range(K):
            pltpu.make_async_copy(g_hbm.at[sidx.at[c0 + b]], rows.at[b],
                                  gsems[b]).wait()
            sds.append(pltpu.async_copy(rows.at[b], acc.at[didx.at[c0 + b]],
                                        ssems[b], add=True))
        for b in range(K):
            sds[b].wait()
            pltpu.async_copy(g_hbm.at[sidx.at[c0 + K + b]], rows.at[b],
                             gsems[b])

    c0 = (ROUNDS - 1) * K
    sds = []
    for b in range(K):
        pltpu.make_async_copy(g_hbm.at[sidx.at[c0 + b]], rows.at[b],
                              gsems[b]).wait()
        sds.append(pltpu.async_copy(rows.at[b], acc.at[didx.at[c0 + b]],
                                    ssems[b], add=True))
    for d in sds:
        d.wait()


_K64 = 8
_C64 = 128
_CHT64 = 40
_K128 = 2
_C128 = 128
_CHT128 = 40


@functools.partial(
    pl.kernel,
    out_type=jax.ShapeDtypeStruct((NC * N_PAD, 64), jnp.float32),
    mesh=_mesh(),
    scratch_types=[
        pltpu.VMEM_SHARED((N_PAD, 64), jnp.float32),
        pltpu.VMEM((_CHT64, _C64), jnp.int32),
        pltpu.VMEM((_CHT64, _C64), jnp.int32),
        pltpu.VMEM((_K64, _C64, 64), jnp.float32),
        pltpu.VMEM((ZR, 64), jnp.float32),
    ] + [pltpu.SemaphoreType.DMA] * (2 * _K64),
    compiler_params=_SC_LINEAR,
)
def _prop64(g_hbm, src_hbm, dst_hbm, out_hbm, acc, sidx, didx, rows, zbuf,
            *sems):
    K = _K64
    cid = lax.axis_index("c")
    sid = lax.axis_index("s")

    @pl.loop(0, ZR)
    def _(r):
        for q in range(4):
            zbuf[r, pl.ds(q * 16, 16)] = jnp.zeros((16,), jnp.float32)

    base = (cid * NS + sid) * EPW
    _stage_idx(src_hbm, dst_hbm, sidx, didx, base, sems[0], _C64, _CHT64)
    zd = [
        pltpu.async_copy(zbuf, acc.at[pl.ds(sid * RPT + z * ZR, ZR)], sems[z])
        for z in range(RPT // ZR)
    ]
    for d in zd:
        d.wait()
    plsc.subcore_barrier()

    _prop_loop(g_hbm, acc, sidx, didx, rows, sems[:K], sems[K:], K, _CHT64)

    plsc.subcore_barrier()
    pltpu.sync_copy(acc.at[pl.ds(sid * RPT, RPT)],
                    out_hbm.at[pl.ds(cid * N_PAD + sid * RPT, RPT)])


@functools.partial(
    pl.kernel,
    out_type=jax.ShapeDtypeStruct((NC * N_PAD, 128), jnp.float32),
    mesh=_mesh(),
    scratch_types=[
        pltpu.VMEM_SHARED((N_PAD, 128), jnp.float32),
        pltpu.VMEM((_CHT128, _C128), jnp.int32),
        pltpu.VMEM((_CHT128, _C128), jnp.int32),
        pltpu.VMEM((_K128, _C128, 128), jnp.float32),
    ] + [pltpu.SemaphoreType.DMA] * (2 * _K128 + 1),
)
def _prop128(g_hbm, src_hbm, dst_hbm, out_hbm, acc, sidx, didx, rows, *sems):
    K = _K128
    gsems, ssems, zsem = sems[:K], sems[K:2 * K], sems[2 * K]
    cid = lax.axis_index("c")
    sid = lax.axis_index("s")

    @pl.loop(0, _C128)
    def _(r):
        for q in range(8):
            rows[0, r, pl.ds(q * 16, 16)] = jnp.zeros((16,), jnp.float32)

    base = (cid * NS + sid) * EPW
    _stage_idx(src_hbm, dst_hbm, sidx, didx, base, zsem, _C128, _CHT128)
    zd = [
        pltpu.async_copy(rows.at[0],
                         acc.at[pl.ds(sid * RPT + z * _C128, _C128)], zsem)
        for z in range(RPT // _C128)
    ]
    for d in zd:
        d.wait()
    plsc.subcore_barrier()

    _prop_loop(g_hbm, acc, sidx, didx, rows, gsems, ssems, K, _CHT128)

    plsc.subcore_barrier()
    pltpu.sync_copy(acc.at[pl.ds(sid * RPT, RPT)],
                    out_hbm.at[pl.ds(cid * N_PAD + sid * RPT, RPT)])


R = 1024
GRID = N_PAD // R


def _tc_first(x, W, c0, c1):
    Din, Dout = W.shape

    def body(x_ref, w_ref, c0_ref, c1_ref, g_ref, dinv_ref):
        h = jnp.dot(x_ref[...], w_ref[...], preferred_element_type=jnp.float32)
        dinv = lax.rsqrt(c0_ref[...] + c1_ref[...] + 1.0)
        dinv_ref[...] = dinv
        g_ref[...] = h * dinv

    return pl.pallas_call(
        body,
        grid=(GRID,),
        in_specs=[
            pl.BlockSpec((R, Din), lambda i: (i, 0)),
            pl.BlockSpec((Din, Dout), lambda i: (0, 0)),
            pl.BlockSpec((R, 1), lambda i: (i, 0)),
            pl.BlockSpec((R, 1), lambda i: (i, 0)),
        ],
        out_specs=[
            pl.BlockSpec((R, Dout), lambda i: (i, 0)),
            pl.BlockSpec((R, 1), lambda i: (i, 0)),
        ],
        out_shape=[
            jax.ShapeDtypeStruct((N, Dout), jnp.float32),
            jax.ShapeDtypeStruct((N_PAD, 1), jnp.float32),
        ],
    )(x, W, c0, c1)


def _tc_mid(acc, g, dinv, b, W):
    Din, Dout = W.shape

    def body(a0_ref, a1_ref, g_ref, dv_ref, b_ref, w_ref, o_ref):
        dcol = dv_ref[...]
        s = dcol * (a0_ref[...] + a1_ref[...] + g_ref[...]) + b_ref[...]
        act = jnp.where(s >= 0, s, 0.2 * s)
        h = jnp.dot(act, w_ref[...], preferred_element_type=jnp.float32)
        o_ref[...] = h * dcol

    return pl.pallas_call(
        body,
        grid=(GRID,),
        in_specs=[
            pl.BlockSpec((R, Din), lambda i: (i, 0)),
            pl.BlockSpec((R, Din), lambda i: (i + GRID, 0)),
            pl.BlockSpec((R, Din), lambda i: (i, 0)),
            pl.BlockSpec((R, 1), lambda i: (i, 0)),
            pl.BlockSpec((1, Din), lambda i: (0, 0)),
            pl.BlockSpec((Din, Dout), lambda i: (0, 0)),
        ],
        out_specs=pl.BlockSpec((R, Dout), lambda i: (i, 0)),
        out_shape=jax.ShapeDtypeStruct((N, Dout), jnp.float32),
    )(acc, acc, g, dinv, b, W)


def _tc_last(acc, g, dinv, b):
    F = g.shape[1]

    def body(a0_ref, a1_ref, g_ref, dv_ref, b_ref, o_ref):
        o_ref[...] = (dv_ref[...] * (a0_ref[...] + a1_ref[...] + g_ref[...])
                      + b_ref[...])

    return pl.pallas_call(
        body,
        grid=(GRID,),
        in_specs=[
            pl.BlockSpec((R, F), lambda i: (i, 0)),
            pl.BlockSpec((R, F), lambda i: (i + GRID, 0)),
            pl.BlockSpec((R, F), lambda i: (i, 0)),
            pl.BlockSpec((R, 1), lambda i: (i, 0)),
            pl.BlockSpec((1, F), lambda i: (0, 0)),
        ],
        out_specs=pl.BlockSpec((R, F), lambda i: (i, 0)),
        out_shape=jax.ShapeDtypeStruct((N, F), jnp.float32),
    )(acc, acc, g, dinv, b)


def kernel(x, edge_index, W1, b1, W2, b2, W3, b3):
    ei = edge_index.astype(jnp.int32)
    src, dst = ei[0], ei[1]

    cnt = _deg(src, dst)
    c0 = cnt[:N_PAD].reshape(N_PAD, 1)
    c1 = cnt[N_PAD:].reshape(N_PAD, 1)

    g1, dinv = _tc_first(x, W1, c0, c1)
    acc = _prop128(g1, src, dst)
    g2 = _tc_mid(acc, g1, dinv, b1.reshape(1, -1), W2)
    acc = _prop64(g2, src, dst)
    g3 = _tc_mid(acc, g2, dinv, b2.reshape(1, -1), W3)
    acc = _prop64(g3, src, dst)
    return _tc_last(acc, g3, dinv, b3.reshape(1, -1))
